# baseline (speedup 1.0000x reference)
import functools

import jax
import jax.numpy as jnp
from jax import lax
from jax.experimental import pallas as pl
from jax.experimental.pallas import tpu as pltpu
from jax.experimental.pallas import tpu_sc as plsc

_NE = 8
_T = 4096
_D = 1024
_H = 2048
_B = 128
_NS = _T * 2 + _NE * _B
_NB = _NS // _B
_HC = 1024
_ROWS = 32
_LANES = 128


def _sigmoid(v):
    return 1.0 / (1.0 + jnp.exp(-v))


def _router_body(x_ref, wg_ref, a1_ref, a2_ref, p1_ref, p2_ref):
    scores = lax.dot_general(
        x_ref[...], wg_ref[...], (((1,), (1,)), ((), ())),
        precision=lax.Precision.DEFAULT,
        preferred_element_type=jnp.float32,
    )
    t = scores.shape[0]
    iota8 = lax.broadcasted_iota(jnp.int32, (t, _NE), 1)
    m1 = jnp.max(scores, axis=1, keepdims=True)
    a1 = jnp.min(jnp.where(scores == m1, iota8, _NE), axis=1, keepdims=True)
    s2 = jnp.where(iota8 == a1, jnp.float32(-jnp.inf), scores)
    m2 = jnp.max(s2, axis=1, keepdims=True)
    a2 = jnp.min(jnp.where(s2 == m2, iota8, _NE), axis=1, keepdims=True)
    p1 = _sigmoid(m1 - m2)
    a1_ref[...] = a1
    a2_ref[...] = a2
    p1_ref[...] = p1
    p2_ref[...] = 1.0 - p1


def _router(xf, Wg):
    return pl.pallas_call(
        _router_body,
        out_shape=[
            jax.ShapeDtypeStruct((_T, 1), jnp.int32),
            jax.ShapeDtypeStruct((_T, 1), jnp.int32),
            jax.ShapeDtypeStruct((_T, 1), jnp.float32),
            jax.ShapeDtypeStruct((_T, 1), jnp.float32),
        ],
    )(xf, Wg)


def _dispatch_body(a1_ref, a2_ref, se_ref, so_ref, be_ref):
    a1 = a1_ref[...]
    a2 = a2_ref[...]
    li = lax.broadcasted_iota(jnp.int32, (_LANES, _LANES), 0)
    lj = lax.broadcasted_iota(jnp.int32, (_LANES, _LANES), 1)
    lt_lane = (li <= lj).astype(jnp.float32)
    ri = lax.broadcasted_iota(jnp.int32, (_ROWS, _ROWS), 0)
    rj = lax.broadcasted_iota(jnp.int32, (_ROWS, _ROWS), 1)
    lt_row_strict = (rj < ri).astype(jnp.float32)

    ranks = []
    counts = []
    for e in range(_NE):
        cnt = ((a1 == e) | (a2 == e)).astype(jnp.float32)
        ic = lax.dot_general(
            cnt, lt_lane, (((1,), (0,)), ((), ())),
            precision=lax.Precision.HIGHEST,
            preferred_element_type=jnp.float32,
        )
        rs = ic[:, _LANES - 1 : _LANES]
        rp = lax.dot_general(
            lt_row_strict, rs, (((1,), (0,)), ((), ())),
            precision=lax.Precision.HIGHEST,
            preferred_element_type=jnp.float32,
        )
        rank = (ic - cnt + rp).astype(jnp.int32)
        ranks.append(rank)
        counts.append(jnp.sum(cnt).astype(jnp.int32))

    offs = []
    off = jnp.int32(0)
    ends_blk = []
    for e in range(_NE):
        offs.append(off)
        padded = ((counts[e] + (_B - 1)) // _B) * _B
        off = off + padded
        ends_blk.append(off // _B)

    se = jnp.zeros_like(a1)
    so = jnp.zeros_like(a1)
    for e in range(_NE):
        slot_e = offs[e] + ranks[e]
        se = jnp.where(a1 == e, slot_e, se)
        so = jnp.where(a2 == e, slot_e, so)
    se_ref[...] = se
    so_ref[...] = so

    bvec = lax.broadcasted_iota(jnp.int32, (1, _LANES), 1)
    be = jnp.zeros((1, _LANES), jnp.int32)
    for e in range(_NE):
        be = be + (bvec >= ends_blk[e]).astype(jnp.int32)
    be_ref[...] = jnp.minimum(be, _NE - 1)


def _dispatch(a1r, a2r):
    return pl.pallas_call(
        _dispatch_body,
        out_shape=[
            jax.ShapeDtypeStruct((_ROWS, _LANES), jnp.int32),
            jax.ShapeDtypeStruct((_ROWS, _LANES), jnp.int32),
            jax.ShapeDtypeStruct((1, _LANES), jnp.int32),
        ],
    )(a1r, a2r)


_SC_CHUNK = 32


def _make_sc_gather():
    mesh = plsc.VectorSubcoreMesh(core_axis_name="c", subcore_axis_name="s")
    info = plsc.get_sparse_core_info()
    nw = info.num_cores * info.num_subcores
    tok_per_w = _T // nw
    nck = tok_per_w // _SC_CHUNK

    @functools.partial(
        pl.kernel,
        mesh=mesh,
        out_type=[
            jax.ShapeDtypeStruct((_NS, _D), jnp.float32),
            jax.ShapeDtypeStruct((_NS, 128), jnp.float32),
        ],
        scratch_types=[
            pltpu.VMEM((_SC_CHUNK,), jnp.int32),
            pltpu.VMEM((_SC_CHUNK,), jnp.int32),
            pltpu.VMEM((_SC_CHUNK,), jnp.float32),
            pltpu.VMEM((_SC_CHUNK,), jnp.float32),
            pltpu.VMEM((_SC_CHUNK, _D), jnp.float32),
            pltpu.VMEM((_SC_CHUNK, 128), jnp.float32),
            pltpu.VMEM((_SC_CHUNK, 128), jnp.float32),
            pltpu.SemaphoreType.DMA,
            pltpu.SemaphoreType.DMA,
            pltpu.SemaphoreType.DMA,
            pltpu.SemaphoreType.DMA,
        ],
    )
    def sc_gather(x_hbm, se_hbm, so_hbm, pe_hbm, po_hbm, xs_hbm, ps_hbm,
                  idxe_v, idxo_v, pe_v, po_v, rows_v, pse_v, pso_v,
                  sem_e, sem_o, sem_pe, sem_po):
        wid = lax.axis_index("s") * info.num_cores + lax.axis_index("c")
        for ck in range(nck):
            base = wid * tok_per_w + ck * _SC_CHUNK
            pltpu.sync_copy(x_hbm.at[pl.ds(base, _SC_CHUNK)], rows_v)
            pltpu.sync_copy(se_hbm.at[pl.ds(base, _SC_CHUNK)], idxe_v)
            pltpu.sync_copy(so_hbm.at[pl.ds(base, _SC_CHUNK)], idxo_v)
            pltpu.sync_copy(pe_hbm.at[pl.ds(base, _SC_CHUNK)], pe_v)
            pltpu.sync_copy(po_hbm.at[pl.ds(base, _SC_CHUNK)], po_v)
            for half in range(_SC_CHUNK // 16):
                pe_reg = pe_v[pl.ds(half * 16, 16)]
                po_reg = po_v[pl.ds(half * 16, 16)]
                for t in range(16):
                    pes = lax.squeeze(lax.slice(pe_reg, (t,), (t + 1,)), (0,))
                    pos = lax.squeeze(lax.slice(po_reg, (t,), (t + 1,)), (0,))
                    pse_v[half * 16 + t, pl.ds(0, 16)] = (
                        lax.broadcast_in_dim(pes, (16,), ()))
                    pso_v[half * 16 + t, pl.ds(0, 16)] = (
                        lax.broadcast_in_dim(pos, (16,), ()))
            cpe = pltpu.async_copy(rows_v, xs_hbm.at[idxe_v], sem_e)
            cpo = pltpu.async_copy(rows_v, xs_hbm.at[idxo_v], sem_o)
            cppe = pltpu.async_copy(pse_v, ps_hbm.at[idxe_v], sem_pe)
            cppo = pltpu.async_copy(pso_v, ps_hbm.at[idxo_v], sem_po)
            cpe.wait()
            cpo.wait()
            cppe.wait()
            cppo.wait()

    return sc_gather


def _moe_body(be_ref, xs_ref, ps_ref, w1_ref, w2_ref, w3_ref, os_ref):
    xbb = xs_ref[...].astype(jnp.bfloat16)
    h1 = lax.dot_general(
        xbb, w1_ref[0], (((1,), (1,)), ((), ())),
        preferred_element_type=jnp.float32,
    )
    h2 = lax.dot_general(
        xbb, w2_ref[0], (((1,), (1,)), ((), ())),
        preferred_element_type=jnp.float32,
    )
    h = (h1 * _sigmoid(h1)) * h2
    eo = lax.dot_general(
        h.astype(jnp.bfloat16), w3_ref[0], (((1,), (1,)), ((), ())),
        preferred_element_type=jnp.float32,
    )
    os_ref[...] = eo * ps_ref[:, 0:1]


def _moe(be, xs, ps, w1b, w2b, w3b):
    grid_spec = pltpu.PrefetchScalarGridSpec(
        num_scalar_prefetch=1,
        grid=(_NB,),
        in_specs=[
            pl.BlockSpec((_B, _D), lambda b, be: (b, 0)),
            pl.BlockSpec((_B, 128), lambda b, be: (b, 0)),
            pl.BlockSpec((1, _H, _D), lambda b, be: (be[b], 0, 0)),
            pl.BlockSpec((1, _H, _D), lambda b, be: (be[b], 0, 0)),
            pl.BlockSpec((1, _D, _H), lambda b, be: (be[b], 0, 0)),
        ],
        out_specs=pl.BlockSpec((_B, _D), lambda b, be: (b, 0)),
    )
    return pl.pallas_call(
        _moe_body,
        grid_spec=grid_spec,
        out_shape=jax.ShapeDtypeStruct((_NS, _D), jnp.float32),
        compiler_params=pltpu.CompilerParams(
            dimension_semantics=("arbitrary",),
        ),
    )(be, xs, ps, w1b, w2b, w3b)


_CB_CHUNK = 32


def _make_sc_combine():
    mesh = plsc.VectorSubcoreMesh(core_axis_name="c", subcore_axis_name="s")
    info = plsc.get_sparse_core_info()
    nw = info.num_cores * info.num_subcores
    tok_per_w = _T // nw
    nck = tok_per_w // _CB_CHUNK

    @functools.partial(
        pl.kernel,
        mesh=mesh,
        out_type=jax.ShapeDtypeStruct((_T, _D), jnp.float32),
        scratch_types=[
            pltpu.VMEM((_CB_CHUNK,), jnp.int32),
            pltpu.VMEM((_CB_CHUNK,), jnp.int32),
            pltpu.VMEM((_CB_CHUNK, _D), jnp.float32),
            pltpu.VMEM((_CB_CHUNK, _D), jnp.float32),
            pltpu.SemaphoreType.DMA,
            pltpu.SemaphoreType.DMA,
        ],
    )
    def sc_combine(os_hbm, se_hbm, so_hbm, out_hbm,
                   idxe_v, idxo_v, re_v, ro_v, sem_e, sem_o):
        wid = lax.axis_index("s") * info.num_cores + lax.axis_index("c")
        for ck in range(nck):
            base = wid * tok_per_w + ck * _CB_CHUNK
            pltpu.sync_copy(se_hbm.at[pl.ds(base, _CB_CHUNK)], idxe_v)
            pltpu.sync_copy(so_hbm.at[pl.ds(base, _CB_CHUNK)], idxo_v)
            cpe = pltpu.async_copy(os_hbm.at[idxe_v], re_v, sem_e)
            cpo = pltpu.async_copy(os_hbm.at[idxo_v], ro_v, sem_o)
            cpe.wait()
            cpo.wait()

            def tbody(t, _):
                def dbody(dd, _):
                    sl = pl.ds(dd * 16, 16)
                    re_v[t, sl] += ro_v[t, sl]
                    return 0

                return lax.fori_loop(0, _D // 16, dbody, 0, unroll=8)

            lax.fori_loop(0, _CB_CHUNK, tbody, 0)
            pltpu.sync_copy(re_v, out_hbm.at[pl.ds(base, _CB_CHUNK)])

    return sc_combine


def kernel(x, Wg, W1, W2, W3):
    b, s, d = x.shape
    xf = x.reshape(b * s, d)
    w1b = W1.astype(jnp.bfloat16)
    w2b = W2.astype(jnp.bfloat16)
    w3b = W3.astype(jnp.bfloat16)

    a1, a2, p1, p2 = _router(xf, Wg)
    a1r = a1.reshape(_ROWS, _LANES)
    a2r = a2.reshape(_ROWS, _LANES)
    se, so, be = _dispatch(a1r, a2r)
    se = se.reshape(_T)
    so = so.reshape(_T)
    be = be.reshape(_LANES)[:_NB]

    xs, ps = _make_sc_gather()(xf, se, so, p1.reshape(_T), p2.reshape(_T))
    os = _moe(be, xs, ps, w1b, w2b, w3b)
    out = _make_sc_combine()(os, se, so)
    return out.reshape(b, s, d)


# speedup vs baseline: 1.4630x; 1.4630x over previous
import functools

import jax
import jax.numpy as jnp
from jax import lax
from jax.experimental import pallas as pl
from jax.experimental.pallas import tpu as pltpu
from jax.experimental.pallas import tpu_sc as plsc

_NE = 8
_T = 4096
_D = 1024
_H = 2048
_B = 256
_NS = _T * 2 + _NE * _B
_NB = _NS // _B
_HC = 1024
_ROWS = 32
_LANES = 128


def _sigmoid(v):
    return 1.0 / (1.0 + jnp.exp(-v))


def _router_body(x_ref, wg_ref, a1_ref, a2_ref, p1_ref, p2_ref):
    scores = lax.dot_general(
        x_ref[...], wg_ref[...], (((1,), (1,)), ((), ())),
        precision=lax.Precision.DEFAULT,
        preferred_element_type=jnp.float32,
    )
    t = scores.shape[0]
    iota8 = lax.broadcasted_iota(jnp.int32, (t, _NE), 1)
    m1 = jnp.max(scores, axis=1, keepdims=True)
    a1 = jnp.min(jnp.where(scores == m1, iota8, _NE), axis=1, keepdims=True)
    s2 = jnp.where(iota8 == a1, jnp.float32(-jnp.inf), scores)
    m2 = jnp.max(s2, axis=1, keepdims=True)
    a2 = jnp.min(jnp.where(s2 == m2, iota8, _NE), axis=1, keepdims=True)
    p1 = _sigmoid(m1 - m2)
    a1_ref[...] = a1
    a2_ref[...] = a2
    p1_ref[...] = p1
    p2_ref[...] = 1.0 - p1


def _router(xf, Wg):
    return pl.pallas_call(
        _router_body,
        out_shape=[
            jax.ShapeDtypeStruct((_T, 1), jnp.int32),
            jax.ShapeDtypeStruct((_T, 1), jnp.int32),
            jax.ShapeDtypeStruct((_T, 1), jnp.float32),
            jax.ShapeDtypeStruct((_T, 1), jnp.float32),
        ],
    )(xf, Wg)


def _dispatch_body(a1_ref, a2_ref, se_ref, so_ref, be_ref):
    a1 = a1_ref[...]
    a2 = a2_ref[...]
    li = lax.broadcasted_iota(jnp.int32, (_LANES, _LANES), 0)
    lj = lax.broadcasted_iota(jnp.int32, (_LANES, _LANES), 1)
    lt_lane = (li <= lj).astype(jnp.float32)
    ri = lax.broadcasted_iota(jnp.int32, (_ROWS, _ROWS), 0)
    rj = lax.broadcasted_iota(jnp.int32, (_ROWS, _ROWS), 1)
    lt_row_strict = (rj < ri).astype(jnp.float32)

    ranks = []
    counts = []
    for e in range(_NE):
        cnt = ((a1 == e) | (a2 == e)).astype(jnp.float32)
        ic = lax.dot_general(
            cnt, lt_lane, (((1,), (0,)), ((), ())),
            precision=lax.Precision.HIGHEST,
            preferred_element_type=jnp.float32,
        )
        rs = ic[:, _LANES - 1 : _LANES]
        rp = lax.dot_general(
            lt_row_strict, rs, (((1,), (0,)), ((), ())),
            precision=lax.Precision.HIGHEST,
            preferred_element_type=jnp.float32,
        )
        rank = (ic - cnt + rp).astype(jnp.int32)
        ranks.append(rank)
        counts.append(jnp.sum(cnt).astype(jnp.int32))

    offs = []
    off = jnp.int32(0)
    ends_blk = []
    for e in range(_NE):
        offs.append(off)
        padded = ((counts[e] + (_B - 1)) // _B) * _B
        off = off + padded
        ends_blk.append(off // _B)

    se = jnp.zeros_like(a1)
    so = jnp.zeros_like(a1)
    for e in range(_NE):
        slot_e = offs[e] + ranks[e]
        se = jnp.where(a1 == e, slot_e, se)
        so = jnp.where(a2 == e, slot_e, so)
    se_ref[...] = se
    so_ref[...] = so

    bvec = lax.broadcasted_iota(jnp.int32, (1, _LANES), 1)
    be = jnp.zeros((1, _LANES), jnp.int32)
    for e in range(_NE):
        be = be + (bvec >= ends_blk[e]).astype(jnp.int32)
    be_ref[...] = jnp.minimum(be, _NE - 1)


def _dispatch(a1r, a2r):
    return pl.pallas_call(
        _dispatch_body,
        out_shape=[
            jax.ShapeDtypeStruct((_ROWS, _LANES), jnp.int32),
            jax.ShapeDtypeStruct((_ROWS, _LANES), jnp.int32),
            jax.ShapeDtypeStruct((1, _LANES), jnp.int32),
        ],
    )(a1r, a2r)


_SC_CHUNK = 32


def _make_sc_gather():
    mesh = plsc.VectorSubcoreMesh(core_axis_name="c", subcore_axis_name="s")
    info = plsc.get_sparse_core_info()
    nw = info.num_cores * info.num_subcores
    tok_per_w = _T // nw
    nck = tok_per_w // _SC_CHUNK

    @functools.partial(
        pl.kernel,
        mesh=mesh,
        out_type=[
            jax.ShapeDtypeStruct((_NS, _D), jnp.float32),
            jax.ShapeDtypeStruct((_NS, 128), jnp.float32),
        ],
        scratch_types=[
            pltpu.VMEM((_SC_CHUNK,), jnp.int32),
            pltpu.VMEM((_SC_CHUNK,), jnp.int32),
            pltpu.VMEM((_SC_CHUNK,), jnp.float32),
            pltpu.VMEM((_SC_CHUNK,), jnp.float32),
            pltpu.VMEM((_SC_CHUNK, _D), jnp.float32),
            pltpu.VMEM((_SC_CHUNK, 128), jnp.float32),
            pltpu.VMEM((_SC_CHUNK, 128), jnp.float32),
            pltpu.SemaphoreType.DMA,
            pltpu.SemaphoreType.DMA,
            pltpu.SemaphoreType.DMA,
            pltpu.SemaphoreType.DMA,
        ],
    )
    def sc_gather(x_hbm, se_hbm, so_hbm, pe_hbm, po_hbm, xs_hbm, ps_hbm,
                  idxe_v, idxo_v, pe_v, po_v, rows_v, pse_v, pso_v,
                  sem_e, sem_o, sem_pe, sem_po):
        wid = lax.axis_index("s") * info.num_cores + lax.axis_index("c")
        for ck in range(nck):
            base = wid * tok_per_w + ck * _SC_CHUNK
            pltpu.sync_copy(x_hbm.at[pl.ds(base, _SC_CHUNK)], rows_v)
            pltpu.sync_copy(se_hbm.at[pl.ds(base, _SC_CHUNK)], idxe_v)
            pltpu.sync_copy(so_hbm.at[pl.ds(base, _SC_CHUNK)], idxo_v)
            pltpu.sync_copy(pe_hbm.at[pl.ds(base, _SC_CHUNK)], pe_v)
            pltpu.sync_copy(po_hbm.at[pl.ds(base, _SC_CHUNK)], po_v)
            for half in range(_SC_CHUNK // 16):
                pe_reg = pe_v[pl.ds(half * 16, 16)]
                po_reg = po_v[pl.ds(half * 16, 16)]
                for t in range(16):
                    pes = lax.squeeze(lax.slice(pe_reg, (t,), (t + 1,)), (0,))
                    pos = lax.squeeze(lax.slice(po_reg, (t,), (t + 1,)), (0,))
                    pse_v[half * 16 + t, pl.ds(0, 16)] = (
                        lax.broadcast_in_dim(pes, (16,), ()))
                    pso_v[half * 16 + t, pl.ds(0, 16)] = (
                        lax.broadcast_in_dim(pos, (16,), ()))
            cpe = pltpu.async_copy(rows_v, xs_hbm.at[idxe_v], sem_e)
            cpo = pltpu.async_copy(rows_v, xs_hbm.at[idxo_v], sem_o)
            cppe = pltpu.async_copy(pse_v, ps_hbm.at[idxe_v], sem_pe)
            cppo = pltpu.async_copy(pso_v, ps_hbm.at[idxo_v], sem_po)
            cpe.wait()
            cpo.wait()
            cppe.wait()
            cppo.wait()

    return sc_gather


def _moe_body(be_ref, xs_ref, ps_ref, w1_ref, w2_ref, w3_ref, os_ref):
    xbb = xs_ref[...].astype(jnp.bfloat16)
    h1 = lax.dot_general(
        xbb, w1_ref[0], (((1,), (1,)), ((), ())),
        preferred_element_type=jnp.float32,
    )
    h2 = lax.dot_general(
        xbb, w2_ref[0], (((1,), (1,)), ((), ())),
        preferred_element_type=jnp.float32,
    )
    h = (h1 * _sigmoid(h1)) * h2
    eo = lax.dot_general(
        h.astype(jnp.bfloat16), w3_ref[0], (((1,), (1,)), ((), ())),
        preferred_element_type=jnp.float32,
    )
    os_ref[...] = eo * ps_ref[:, 0:1]


def _moe(be, xs, ps, w1b, w2b, w3b):
    grid_spec = pltpu.PrefetchScalarGridSpec(
        num_scalar_prefetch=1,
        grid=(_NB,),
        in_specs=[
            pl.BlockSpec((_B, _D), lambda b, be: (b, 0)),
            pl.BlockSpec((_B, 128), lambda b, be: (b, 0)),
            pl.BlockSpec((1, _H, _D), lambda b, be: (be[b], 0, 0)),
            pl.BlockSpec((1, _H, _D), lambda b, be: (be[b], 0, 0)),
            pl.BlockSpec((1, _D, _H), lambda b, be: (be[b], 0, 0)),
        ],
        out_specs=pl.BlockSpec((_B, _D), lambda b, be: (b, 0)),
    )
    return pl.pallas_call(
        _moe_body,
        grid_spec=grid_spec,
        out_shape=jax.ShapeDtypeStruct((_NS, _D), jnp.float32),
        compiler_params=pltpu.CompilerParams(
            dimension_semantics=("arbitrary",),
        ),
    )(be, xs, ps, w1b, w2b, w3b)


_CB_CHUNK = 32


def _make_sc_combine():
    mesh = plsc.VectorSubcoreMesh(core_axis_name="c", subcore_axis_name="s")
    info = plsc.get_sparse_core_info()
    nw = info.num_cores * info.num_subcores
    tok_per_w = _T // nw
    nck = tok_per_w // _CB_CHUNK

    @functools.partial(
        pl.kernel,
        mesh=mesh,
        out_type=jax.ShapeDtypeStruct((_T, _D), jnp.float32),
        scratch_types=[
            pltpu.VMEM((_CB_CHUNK,), jnp.int32),
            pltpu.VMEM((_CB_CHUNK,), jnp.int32),
            pltpu.VMEM((_CB_CHUNK, _D), jnp.float32),
            pltpu.VMEM((_CB_CHUNK, _D), jnp.float32),
            pltpu.SemaphoreType.DMA,
            pltpu.SemaphoreType.DMA,
        ],
    )
    def sc_combine(os_hbm, se_hbm, so_hbm, out_hbm,
                   idxe_v, idxo_v, re_v, ro_v, sem_e, sem_o):
        wid = lax.axis_index("s") * info.num_cores + lax.axis_index("c")
        for ck in range(nck):
            base = wid * tok_per_w + ck * _CB_CHUNK
            pltpu.sync_copy(se_hbm.at[pl.ds(base, _CB_CHUNK)], idxe_v)
            pltpu.sync_copy(so_hbm.at[pl.ds(base, _CB_CHUNK)], idxo_v)
            cpe = pltpu.async_copy(os_hbm.at[idxe_v], re_v, sem_e)
            cpo = pltpu.async_copy(os_hbm.at[idxo_v], ro_v, sem_o)
            cpe.wait()
            cpo.wait()

            def tbody(t, _):
                def dbody(dd, _):
                    sl = pl.ds(dd * 16, 16)
                    re_v[t, sl] += ro_v[t, sl]
                    return 0

                return lax.fori_loop(0, _D // 16, dbody, 0, unroll=8)

            lax.fori_loop(0, _CB_CHUNK, tbody, 0)
            pltpu.sync_copy(re_v, out_hbm.at[pl.ds(base, _CB_CHUNK)])

    return sc_combine


def kernel(x, Wg, W1, W2, W3):
    b, s, d = x.shape
    xf = x.reshape(b * s, d)
    w1b = W1.astype(jnp.bfloat16)
    w2b = W2.astype(jnp.bfloat16)
    w3b = W3.astype(jnp.bfloat16)

    a1, a2, p1, p2 = _router(xf, Wg)
    a1r = a1.reshape(_ROWS, _LANES)
    a2r = a2.reshape(_ROWS, _LANES)
    se, so, be = _dispatch(a1r, a2r)
    se = se.reshape(_T)
    so = so.reshape(_T)
    be = be.reshape(_LANES)[:_NB]

    xs, ps = _make_sc_gather()(xf, se, so, p1.reshape(_T), p2.reshape(_T))
    os = _moe(be, xs, ps, w1b, w2b, w3b)
    out = _make_sc_combine()(os, se, so)
    return out.reshape(b, s, d)


# speedup vs baseline: 1.4814x; 1.0126x over previous
import functools

import jax
import jax.numpy as jnp
from jax import lax
from jax.experimental import pallas as pl
from jax.experimental.pallas import tpu as pltpu
from jax.experimental.pallas import tpu_sc as plsc

_NE = 8
_T = 4096
_D = 1024
_H = 2048
_B = 256
_NS = _T * 2 + _NE * _B
_NB = _NS // _B
_HC = 1024
_ROWS = 32
_LANES = 128


def _sigmoid(v):
    return 1.0 / (1.0 + jnp.exp(-v))


def _router_body(x_ref, wg_ref, a1_ref, a2_ref, p1_ref, p2_ref):
    scores = lax.dot_general(
        x_ref[...], wg_ref[...], (((1,), (1,)), ((), ())),
        precision=lax.Precision.DEFAULT,
        preferred_element_type=jnp.float32,
    )
    t = scores.shape[0]
    iota8 = lax.broadcasted_iota(jnp.int32, (t, _NE), 1)
    m1 = jnp.max(scores, axis=1, keepdims=True)
    a1 = jnp.min(jnp.where(scores == m1, iota8, _NE), axis=1, keepdims=True)
    s2 = jnp.where(iota8 == a1, jnp.float32(-jnp.inf), scores)
    m2 = jnp.max(s2, axis=1, keepdims=True)
    a2 = jnp.min(jnp.where(s2 == m2, iota8, _NE), axis=1, keepdims=True)
    p1 = _sigmoid(m1 - m2)
    a1_ref[...] = a1
    a2_ref[...] = a2
    p1_ref[...] = p1
    p2_ref[...] = 1.0 - p1


def _router(xf, Wg):
    return pl.pallas_call(
        _router_body,
        out_shape=[
            jax.ShapeDtypeStruct((_T, 1), jnp.int32),
            jax.ShapeDtypeStruct((_T, 1), jnp.int32),
            jax.ShapeDtypeStruct((_T, 1), jnp.float32),
            jax.ShapeDtypeStruct((_T, 1), jnp.float32),
        ],
    )(xf, Wg)


def _dispatch_body(a1_ref, a2_ref, se_ref, so_ref, be_ref):
    a1 = a1_ref[...]
    a2 = a2_ref[...]
    li = lax.broadcasted_iota(jnp.int32, (_LANES, _LANES), 0)
    lj = lax.broadcasted_iota(jnp.int32, (_LANES, _LANES), 1)
    lt_lane = (li <= lj).astype(jnp.float32)
    ri = lax.broadcasted_iota(jnp.int32, (_ROWS, _ROWS), 0)
    rj = lax.broadcasted_iota(jnp.int32, (_ROWS, _ROWS), 1)
    lt_row_strict = (rj < ri).astype(jnp.float32)

    ranks = []
    counts = []
    for e in range(_NE):
        cnt = ((a1 == e) | (a2 == e)).astype(jnp.float32)
        ic = lax.dot_general(
            cnt, lt_lane, (((1,), (0,)), ((), ())),
            precision=lax.Precision.HIGHEST,
            preferred_element_type=jnp.float32,
        )
        rs = ic[:, _LANES - 1 : _LANES]
        rp = lax.dot_general(
            lt_row_strict, rs, (((1,), (0,)), ((), ())),
            precision=lax.Precision.HIGHEST,
            preferred_element_type=jnp.float32,
        )
        rank = (ic - cnt + rp).astype(jnp.int32)
        ranks.append(rank)
        counts.append(jnp.sum(cnt).astype(jnp.int32))

    offs = []
    off = jnp.int32(0)
    ends_blk = []
    for e in range(_NE):
        offs.append(off)
        padded = ((counts[e] + (_B - 1)) // _B) * _B
        off = off + padded
        ends_blk.append(off // _B)

    se = jnp.zeros_like(a1)
    so = jnp.zeros_like(a1)
    for e in range(_NE):
        slot_e = offs[e] + ranks[e]
        se = jnp.where(a1 == e, slot_e, se)
        so = jnp.where(a2 == e, slot_e, so)
    se_ref[...] = se
    so_ref[...] = so

    bvec = lax.broadcasted_iota(jnp.int32, (1, _LANES), 1)
    be = jnp.zeros((1, _LANES), jnp.int32)
    for e in range(_NE):
        be = be + (bvec >= ends_blk[e]).astype(jnp.int32)
    be_ref[...] = jnp.minimum(be, _NE - 1)


def _dispatch(a1r, a2r):
    return pl.pallas_call(
        _dispatch_body,
        out_shape=[
            jax.ShapeDtypeStruct((_ROWS, _LANES), jnp.int32),
            jax.ShapeDtypeStruct((_ROWS, _LANES), jnp.int32),
            jax.ShapeDtypeStruct((1, _LANES), jnp.int32),
        ],
    )(a1r, a2r)


_SC_CHUNK = 32


def _make_sc_gather():
    mesh = plsc.VectorSubcoreMesh(core_axis_name="c", subcore_axis_name="s")
    info = plsc.get_sparse_core_info()
    nw = info.num_cores * info.num_subcores
    tok_per_w = _T // nw
    nck = tok_per_w // _SC_CHUNK

    @functools.partial(
        pl.kernel,
        mesh=mesh,
        out_type=[
            jax.ShapeDtypeStruct((_NS, _D), jnp.float32),
            jax.ShapeDtypeStruct((_NS, 128), jnp.float32),
        ],
        scratch_types=[
            pltpu.VMEM((2, _SC_CHUNK), jnp.int32),
            pltpu.VMEM((2, _SC_CHUNK), jnp.int32),
            pltpu.VMEM((_SC_CHUNK,), jnp.float32),
            pltpu.VMEM((_SC_CHUNK,), jnp.float32),
            pltpu.VMEM((2, _SC_CHUNK, _D), jnp.float32),
            pltpu.VMEM((2, _SC_CHUNK, 128), jnp.float32),
            pltpu.VMEM((2, _SC_CHUNK, 128), jnp.float32),
            pltpu.SemaphoreType.DMA,
            pltpu.SemaphoreType.DMA,
            pltpu.SemaphoreType.DMA,
            pltpu.SemaphoreType.DMA,
            pltpu.SemaphoreType.DMA,
            pltpu.SemaphoreType.DMA,
            pltpu.SemaphoreType.DMA,
            pltpu.SemaphoreType.DMA,
        ],
    )
    def sc_gather(x_hbm, se_hbm, so_hbm, pe_hbm, po_hbm, xs_hbm, ps_hbm,
                  idxe_v, idxo_v, pe_v, po_v, rows_v, pse_v, pso_v,
                  *sems):
        wid = lax.axis_index("s") * info.num_cores + lax.axis_index("c")
        pend = [None, None]
        for ck in range(nck):
            sl = ck % 2
            base = wid * tok_per_w + ck * _SC_CHUNK
            if pend[sl] is not None:
                for cp in pend[sl]:
                    cp.wait()
                pend[sl] = None
            pltpu.sync_copy(x_hbm.at[pl.ds(base, _SC_CHUNK)], rows_v.at[sl])
            pltpu.sync_copy(se_hbm.at[pl.ds(base, _SC_CHUNK)], idxe_v.at[sl])
            pltpu.sync_copy(so_hbm.at[pl.ds(base, _SC_CHUNK)], idxo_v.at[sl])
            pltpu.sync_copy(pe_hbm.at[pl.ds(base, _SC_CHUNK)], pe_v)
            pltpu.sync_copy(po_hbm.at[pl.ds(base, _SC_CHUNK)], po_v)
            for half in range(_SC_CHUNK // 16):
                pe_reg = pe_v[pl.ds(half * 16, 16)]
                po_reg = po_v[pl.ds(half * 16, 16)]
                for t in range(16):
                    pes = lax.squeeze(lax.slice(pe_reg, (t,), (t + 1,)), (0,))
                    pos = lax.squeeze(lax.slice(po_reg, (t,), (t + 1,)), (0,))
                    pse_v[sl, half * 16 + t, pl.ds(0, 16)] = (
                        lax.broadcast_in_dim(pes, (16,), ()))
                    pso_v[sl, half * 16 + t, pl.ds(0, 16)] = (
                        lax.broadcast_in_dim(pos, (16,), ()))
            pend[sl] = [
                pltpu.async_copy(
                    rows_v.at[sl], xs_hbm.at[idxe_v.at[sl]], sems[4 * sl]),
                pltpu.async_copy(
                    rows_v.at[sl], xs_hbm.at[idxo_v.at[sl]], sems[4 * sl + 1]),
                pltpu.async_copy(
                    pse_v.at[sl], ps_hbm.at[idxe_v.at[sl]], sems[4 * sl + 2]),
                pltpu.async_copy(
                    pso_v.at[sl], ps_hbm.at[idxo_v.at[sl]], sems[4 * sl + 3]),
            ]
        for pd in pend:
            if pd is not None:
                for cp in pd:
                    cp.wait()

    return sc_gather


def _moe_body(be_ref, xs_ref, ps_ref, w1_ref, w2_ref, w3_ref, os_ref):
    xbb = xs_ref[...].astype(jnp.bfloat16)
    h1 = lax.dot_general(
        xbb, w1_ref[0], (((1,), (1,)), ((), ())),
        preferred_element_type=jnp.float32,
    )
    h2 = lax.dot_general(
        xbb, w2_ref[0], (((1,), (1,)), ((), ())),
        preferred_element_type=jnp.float32,
    )
    h = (h1 * _sigmoid(h1)) * h2
    eo = lax.dot_general(
        h.astype(jnp.bfloat16), w3_ref[0], (((1,), (1,)), ((), ())),
        preferred_element_type=jnp.float32,
    )
    os_ref[...] = eo * ps_ref[:, 0:1]


def _moe(be, xs, ps, w1b, w2b, w3b):
    grid_spec = pltpu.PrefetchScalarGridSpec(
        num_scalar_prefetch=1,
        grid=(_NB,),
        in_specs=[
            pl.BlockSpec((_B, _D), lambda b, be: (b, 0)),
            pl.BlockSpec((_B, 128), lambda b, be: (b, 0)),
            pl.BlockSpec((1, _H, _D), lambda b, be: (be[b], 0, 0)),
            pl.BlockSpec((1, _H, _D), lambda b, be: (be[b], 0, 0)),
            pl.BlockSpec((1, _D, _H), lambda b, be: (be[b], 0, 0)),
        ],
        out_specs=pl.BlockSpec((_B, _D), lambda b, be: (b, 0)),
    )
    return pl.pallas_call(
        _moe_body,
        grid_spec=grid_spec,
        out_shape=jax.ShapeDtypeStruct((_NS, _D), jnp.float32),
        compiler_params=pltpu.CompilerParams(
            dimension_semantics=("arbitrary",),
        ),
    )(be, xs, ps, w1b, w2b, w3b)


_CB_CHUNK = 16


def _make_sc_combine():
    mesh = plsc.VectorSubcoreMesh(core_axis_name="c", subcore_axis_name="s")
    info = plsc.get_sparse_core_info()
    nw = info.num_cores * info.num_subcores
    tok_per_w = _T // nw
    nck = tok_per_w // _CB_CHUNK

    @functools.partial(
        pl.kernel,
        mesh=mesh,
        out_type=jax.ShapeDtypeStruct((_T, _D), jnp.float32),
        scratch_types=[
            pltpu.VMEM((2, _CB_CHUNK), jnp.int32),
            pltpu.VMEM((2, _CB_CHUNK), jnp.int32),
            pltpu.VMEM((2, _CB_CHUNK, _D), jnp.float32),
            pltpu.VMEM((2, _CB_CHUNK, _D), jnp.float32),
            pltpu.SemaphoreType.DMA,
            pltpu.SemaphoreType.DMA,
            pltpu.SemaphoreType.DMA,
            pltpu.SemaphoreType.DMA,
        ],
    )
    def sc_combine(os_hbm, se_hbm, so_hbm, out_hbm,
                   idxe_v, idxo_v, re_v, ro_v, *sems):
        wid = lax.axis_index("s") * info.num_cores + lax.axis_index("c")

        def issue(ck):
            sl = ck % 2
            base = wid * tok_per_w + ck * _CB_CHUNK
            pltpu.sync_copy(se_hbm.at[pl.ds(base, _CB_CHUNK)], idxe_v.at[sl])
            pltpu.sync_copy(so_hbm.at[pl.ds(base, _CB_CHUNK)], idxo_v.at[sl])
            return [
                pltpu.async_copy(
                    os_hbm.at[idxe_v.at[sl]], re_v.at[sl], sems[2 * sl]),
                pltpu.async_copy(
                    os_hbm.at[idxo_v.at[sl]], ro_v.at[sl], sems[2 * sl + 1]),
            ]

        pend = issue(0)
        for ck in range(nck):
            sl = ck % 2
            for cp in pend:
                cp.wait()
            if ck + 1 < nck:
                pend = issue(ck + 1)

            def tbody(t, _, sl=sl):
                def dbody(dd, _):
                    ds16 = pl.ds(dd * 16, 16)
                    re_v[sl, t, ds16] += ro_v[sl, t, ds16]
                    return 0

                return lax.fori_loop(0, _D // 16, dbody, 0, unroll=8)

            lax.fori_loop(0, _CB_CHUNK, tbody, 0)
            base = wid * tok_per_w + ck * _CB_CHUNK
            pltpu.sync_copy(re_v.at[sl], out_hbm.at[pl.ds(base, _CB_CHUNK)])

    return sc_combine


def kernel(x, Wg, W1, W2, W3):
    b, s, d = x.shape
    xf = x.reshape(b * s, d)
    w1b = W1.astype(jnp.bfloat16)
    w2b = W2.astype(jnp.bfloat16)
    w3b = W3.astype(jnp.bfloat16)

    a1, a2, p1, p2 = _router(xf, Wg)
    a1r = a1.reshape(_ROWS, _LANES)
    a2r = a2.reshape(_ROWS, _LANES)
    se, so, be = _dispatch(a1r, a2r)
    se = se.reshape(_T)
    so = so.reshape(_T)
    be = be.reshape(_LANES)[:_NB]

    xs, ps = _make_sc_gather()(xf, se, so, p1.reshape(_T), p2.reshape(_T))
    os = _moe(be, xs, ps, w1b, w2b, w3b)
    out = _make_sc_combine()(os, se, so)
    return out.reshape(b, s, d)


# speedup vs baseline: 1.7215x; 1.1621x over previous
import functools

import jax
import jax.numpy as jnp
from jax import lax
from jax.experimental import pallas as pl
from jax.experimental.pallas import tpu as pltpu
from jax.experimental.pallas import tpu_sc as plsc

_NE = 8
_T = 4096
_D = 1024
_H = 2048
_B = 256
_NS = _T * 2 + _NE * _B
_NB = _NS // _B
_HC = 1024
_ROWS = 32
_LANES = 128


def _sigmoid(v):
    return 1.0 / (1.0 + jnp.exp(-v))


def _router_body(x_ref, wg_ref, a1_ref, a2_ref, p1_ref, p2_ref):
    scores = lax.dot_general(
        x_ref[...], wg_ref[...], (((1,), (1,)), ((), ())),
        precision=lax.Precision.DEFAULT,
        preferred_element_type=jnp.float32,
    )
    t = scores.shape[0]
    iota8 = lax.broadcasted_iota(jnp.int32, (t, _NE), 1)
    m1 = jnp.max(scores, axis=1, keepdims=True)
    a1 = jnp.min(jnp.where(scores == m1, iota8, _NE), axis=1, keepdims=True)
    s2 = jnp.where(iota8 == a1, jnp.float32(-jnp.inf), scores)
    m2 = jnp.max(s2, axis=1, keepdims=True)
    a2 = jnp.min(jnp.where(s2 == m2, iota8, _NE), axis=1, keepdims=True)
    p1 = _sigmoid(m1 - m2)
    a1_ref[...] = a1
    a2_ref[...] = a2
    p1_ref[...] = p1
    p2_ref[...] = 1.0 - p1


def _router(xf, Wg):
    return pl.pallas_call(
        _router_body,
        out_shape=[
            jax.ShapeDtypeStruct((_T, 1), jnp.int32),
            jax.ShapeDtypeStruct((_T, 1), jnp.int32),
            jax.ShapeDtypeStruct((_T, 1), jnp.float32),
            jax.ShapeDtypeStruct((_T, 1), jnp.float32),
        ],
    )(xf, Wg)


def _dispatch_body(a1_ref, a2_ref, se_ref, so_ref, be_ref):
    a1 = a1_ref[...]
    a2 = a2_ref[...]
    li = lax.broadcasted_iota(jnp.int32, (_LANES, _LANES), 0)
    lj = lax.broadcasted_iota(jnp.int32, (_LANES, _LANES), 1)
    lt_lane = (li <= lj).astype(jnp.float32)
    ri = lax.broadcasted_iota(jnp.int32, (_ROWS, _ROWS), 0)
    rj = lax.broadcasted_iota(jnp.int32, (_ROWS, _ROWS), 1)
    lt_row_strict = (rj < ri).astype(jnp.float32)

    ranks = []
    counts = []
    for e in range(_NE):
        cnt = ((a1 == e) | (a2 == e)).astype(jnp.float32)
        ic = lax.dot_general(
            cnt, lt_lane, (((1,), (0,)), ((), ())),
            precision=lax.Precision.HIGHEST,
            preferred_element_type=jnp.float32,
        )
        rs = ic[:, _LANES - 1 : _LANES]
        rp = lax.dot_general(
            lt_row_strict, rs, (((1,), (0,)), ((), ())),
            precision=lax.Precision.HIGHEST,
            preferred_element_type=jnp.float32,
        )
        rank = (ic - cnt + rp).astype(jnp.int32)
        ranks.append(rank)
        counts.append(jnp.sum(cnt).astype(jnp.int32))

    offs = []
    off = jnp.int32(0)
    ends_blk = []
    for e in range(_NE):
        offs.append(off)
        padded = ((counts[e] + (_B - 1)) // _B) * _B
        off = off + padded
        ends_blk.append(off // _B)

    se = jnp.zeros_like(a1)
    so = jnp.zeros_like(a1)
    for e in range(_NE):
        slot_e = offs[e] + ranks[e]
        se = jnp.where(a1 == e, slot_e, se)
        so = jnp.where(a2 == e, slot_e, so)
    se_ref[...] = se
    so_ref[...] = so

    bvec = lax.broadcasted_iota(jnp.int32, (1, _LANES), 1)
    be = jnp.zeros((1, _LANES), jnp.int32)
    for e in range(_NE):
        be = be + (bvec >= ends_blk[e]).astype(jnp.int32)
    be_ref[...] = jnp.minimum(be, _NE - 1)


def _dispatch(a1r, a2r):
    return pl.pallas_call(
        _dispatch_body,
        out_shape=[
            jax.ShapeDtypeStruct((_ROWS, _LANES), jnp.int32),
            jax.ShapeDtypeStruct((_ROWS, _LANES), jnp.int32),
            jax.ShapeDtypeStruct((1, _LANES), jnp.int32),
        ],
    )(a1r, a2r)


_SC_CHUNK = 32


def _make_sc_gather():
    mesh = plsc.VectorSubcoreMesh(core_axis_name="c", subcore_axis_name="s")
    info = plsc.get_sparse_core_info()
    nw = info.num_cores * info.num_subcores
    tok_per_w = _T // nw
    nck = tok_per_w // _SC_CHUNK

    @functools.partial(
        pl.kernel,
        mesh=mesh,
        out_type=[
            jax.ShapeDtypeStruct((_NS, _D), jnp.float32),
            jax.ShapeDtypeStruct((_NS, 128), jnp.float32),
        ],
        scratch_types=[
            pltpu.VMEM((2, _SC_CHUNK), jnp.int32),
            pltpu.VMEM((2, _SC_CHUNK), jnp.int32),
            pltpu.VMEM((_SC_CHUNK,), jnp.float32),
            pltpu.VMEM((_SC_CHUNK,), jnp.float32),
            pltpu.VMEM((2, _SC_CHUNK, _D), jnp.float32),
            pltpu.VMEM((2, _SC_CHUNK, 128), jnp.float32),
            pltpu.VMEM((2, _SC_CHUNK, 128), jnp.float32),
            pltpu.SemaphoreType.DMA,
            pltpu.SemaphoreType.DMA,
            pltpu.SemaphoreType.DMA,
            pltpu.SemaphoreType.DMA,
            pltpu.SemaphoreType.DMA,
            pltpu.SemaphoreType.DMA,
            pltpu.SemaphoreType.DMA,
            pltpu.SemaphoreType.DMA,
        ],
    )
    def sc_gather(x_hbm, se_hbm, so_hbm, pe_hbm, po_hbm, xs_hbm, ps_hbm,
                  idxe_v, idxo_v, pe_v, po_v, rows_v, pse_v, pso_v,
                  *sems):
        wid = lax.axis_index("s") * info.num_cores + lax.axis_index("c")
        pend = [None, None]
        for ck in range(nck):
            sl = ck % 2
            base = wid * tok_per_w + ck * _SC_CHUNK
            if pend[sl] is not None:
                for cp in pend[sl]:
                    cp.wait()
                pend[sl] = None
            pltpu.sync_copy(x_hbm.at[pl.ds(base, _SC_CHUNK)], rows_v.at[sl])
            pltpu.sync_copy(se_hbm.at[pl.ds(base, _SC_CHUNK)], idxe_v.at[sl])
            pltpu.sync_copy(so_hbm.at[pl.ds(base, _SC_CHUNK)], idxo_v.at[sl])
            pltpu.sync_copy(pe_hbm.at[pl.ds(base, _SC_CHUNK)], pe_v)
            pltpu.sync_copy(po_hbm.at[pl.ds(base, _SC_CHUNK)], po_v)
            for half in range(_SC_CHUNK // 16):
                pe_reg = pe_v[pl.ds(half * 16, 16)]
                po_reg = po_v[pl.ds(half * 16, 16)]
                for t in range(16):
                    pes = lax.squeeze(lax.slice(pe_reg, (t,), (t + 1,)), (0,))
                    pos = lax.squeeze(lax.slice(po_reg, (t,), (t + 1,)), (0,))
                    pse_v[sl, half * 16 + t, pl.ds(0, 16)] = (
                        lax.broadcast_in_dim(pes, (16,), ()))
                    pso_v[sl, half * 16 + t, pl.ds(0, 16)] = (
                        lax.broadcast_in_dim(pos, (16,), ()))
            pend[sl] = [
                pltpu.async_copy(
                    rows_v.at[sl], xs_hbm.at[idxe_v.at[sl]], sems[4 * sl]),
                pltpu.async_copy(
                    rows_v.at[sl], xs_hbm.at[idxo_v.at[sl]], sems[4 * sl + 1]),
                pltpu.async_copy(
                    pse_v.at[sl], ps_hbm.at[idxe_v.at[sl]], sems[4 * sl + 2]),
                pltpu.async_copy(
                    pso_v.at[sl], ps_hbm.at[idxo_v.at[sl]], sems[4 * sl + 3]),
            ]
        for pd in pend:
            if pd is not None:
                for cp in pd:
                    cp.wait()

    return sc_gather


def _moe_body(be_ref, xs_ref, ps_ref, w1_ref, w2_ref, w3_ref, os_ref):
    xbb = xs_ref[...].astype(jnp.bfloat16)
    h1 = lax.dot_general(
        xbb, w1_ref[0].astype(jnp.bfloat16), (((1,), (1,)), ((), ())),
        preferred_element_type=jnp.float32,
    )
    h2 = lax.dot_general(
        xbb, w2_ref[0].astype(jnp.bfloat16), (((1,), (1,)), ((), ())),
        preferred_element_type=jnp.float32,
    )
    h = (h1 * _sigmoid(h1)) * h2
    eo = lax.dot_general(
        h.astype(jnp.bfloat16), w3_ref[0].astype(jnp.bfloat16),
        (((1,), (1,)), ((), ())),
        preferred_element_type=jnp.float32,
    )
    os_ref[...] = eo * ps_ref[:, 0:1]


def _moe(be, xs, ps, w1b, w2b, w3b):
    grid_spec = pltpu.PrefetchScalarGridSpec(
        num_scalar_prefetch=1,
        grid=(_NB,),
        in_specs=[
            pl.BlockSpec((_B, _D), lambda b, be: (b, 0)),
            pl.BlockSpec((_B, 128), lambda b, be: (b, 0)),
            pl.BlockSpec((1, _H, _D), lambda b, be: (be[b], 0, 0)),
            pl.BlockSpec((1, _H, _D), lambda b, be: (be[b], 0, 0)),
            pl.BlockSpec((1, _D, _H), lambda b, be: (be[b], 0, 0)),
        ],
        out_specs=pl.BlockSpec((_B, _D), lambda b, be: (b, 0)),
    )
    return pl.pallas_call(
        _moe_body,
        grid_spec=grid_spec,
        out_shape=jax.ShapeDtypeStruct((_NS, _D), jnp.float32),
        compiler_params=pltpu.CompilerParams(
            dimension_semantics=("arbitrary",),
        ),
    )(be, xs, ps, w1b, w2b, w3b)


_CB_CHUNK = 16


def _make_sc_combine():
    mesh = plsc.VectorSubcoreMesh(core_axis_name="c", subcore_axis_name="s")
    info = plsc.get_sparse_core_info()
    nw = info.num_cores * info.num_subcores
    tok_per_w = _T // nw
    nck = tok_per_w // _CB_CHUNK

    @functools.partial(
        pl.kernel,
        mesh=mesh,
        out_type=jax.ShapeDtypeStruct((_T, _D), jnp.float32),
        scratch_types=[
            pltpu.VMEM((2, _CB_CHUNK), jnp.int32),
            pltpu.VMEM((2, _CB_CHUNK), jnp.int32),
            pltpu.VMEM((2, _CB_CHUNK, _D), jnp.float32),
            pltpu.VMEM((2, _CB_CHUNK, _D), jnp.float32),
            pltpu.SemaphoreType.DMA,
            pltpu.SemaphoreType.DMA,
            pltpu.SemaphoreType.DMA,
            pltpu.SemaphoreType.DMA,
        ],
    )
    def sc_combine(os_hbm, se_hbm, so_hbm, out_hbm,
                   idxe_v, idxo_v, re_v, ro_v, *sems):
        wid = lax.axis_index("s") * info.num_cores + lax.axis_index("c")

        def issue(ck):
            sl = ck % 2
            base = wid * tok_per_w + ck * _CB_CHUNK
            pltpu.sync_copy(se_hbm.at[pl.ds(base, _CB_CHUNK)], idxe_v.at[sl])
            pltpu.sync_copy(so_hbm.at[pl.ds(base, _CB_CHUNK)], idxo_v.at[sl])
            return [
                pltpu.async_copy(
                    os_hbm.at[idxe_v.at[sl]], re_v.at[sl], sems[2 * sl]),
                pltpu.async_copy(
                    os_hbm.at[idxo_v.at[sl]], ro_v.at[sl], sems[2 * sl + 1]),
            ]

        pend = issue(0)
        for ck in range(nck):
            sl = ck % 2
            for cp in pend:
                cp.wait()
            if ck + 1 < nck:
                pend = issue(ck + 1)

            def tbody(t, _, sl=sl):
                def dbody(dd, _):
                    ds16 = pl.ds(dd * 16, 16)
                    re_v[sl, t, ds16] += ro_v[sl, t, ds16]
                    return 0

                return lax.fori_loop(0, _D // 16, dbody, 0, unroll=8)

            lax.fori_loop(0, _CB_CHUNK, tbody, 0)
            base = wid * tok_per_w + ck * _CB_CHUNK
            pltpu.sync_copy(re_v.at[sl], out_hbm.at[pl.ds(base, _CB_CHUNK)])

    return sc_combine


def kernel(x, Wg, W1, W2, W3):
    b, s, d = x.shape
    xf = x.reshape(b * s, d)

    a1, a2, p1, p2 = _router(xf, Wg)
    a1r = a1.reshape(_ROWS, _LANES)
    a2r = a2.reshape(_ROWS, _LANES)
    se, so, be = _dispatch(a1r, a2r)
    se = se.reshape(_T)
    so = so.reshape(_T)
    be = be.reshape(_LANES)[:_NB]

    xs, ps = _make_sc_gather()(xf, se, so, p1.reshape(_T), p2.reshape(_T))
    os = _moe(be, xs, ps, W1, W2, W3)
    out = _make_sc_combine()(os, se, so)
    return out.reshape(b, s, d)


# speedup vs baseline: 1.8412x; 1.0695x over previous
import functools

import jax
import jax.numpy as jnp
from jax import lax
from jax.experimental import pallas as pl
from jax.experimental.pallas import tpu as pltpu
from jax.experimental.pallas import tpu_sc as plsc

_NE = 8
_T = 4096
_D = 1024
_H = 2048
_B = 256
_NS = _T * 2 + _NE * _B
_NB = _NS // _B
_HC = 1024
_ROWS = 32
_LANES = 128


def _sigmoid(v):
    return 1.0 / (1.0 + jnp.exp(-v))


def _router_body(x_ref, wg_ref, a1_ref, a2_ref, p1_ref, p2_ref):
    scores = lax.dot_general(
        x_ref[...], wg_ref[...], (((1,), (1,)), ((), ())),
        precision=lax.Precision.DEFAULT,
        preferred_element_type=jnp.float32,
    )
    t = scores.shape[0]
    iota8 = lax.broadcasted_iota(jnp.int32, (t, _NE), 1)
    m1 = jnp.max(scores, axis=1, keepdims=True)
    a1 = jnp.min(jnp.where(scores == m1, iota8, _NE), axis=1, keepdims=True)
    s2 = jnp.where(iota8 == a1, jnp.float32(-jnp.inf), scores)
    m2 = jnp.max(s2, axis=1, keepdims=True)
    a2 = jnp.min(jnp.where(s2 == m2, iota8, _NE), axis=1, keepdims=True)
    p1 = _sigmoid(m1 - m2)
    a1_ref[...] = a1
    a2_ref[...] = a2
    p1_ref[...] = p1
    p2_ref[...] = 1.0 - p1


def _router(xf, Wg):
    return pl.pallas_call(
        _router_body,
        out_shape=[
            jax.ShapeDtypeStruct((_T, 1), jnp.int32),
            jax.ShapeDtypeStruct((_T, 1), jnp.int32),
            jax.ShapeDtypeStruct((_T, 1), jnp.float32),
            jax.ShapeDtypeStruct((_T, 1), jnp.float32),
        ],
    )(xf, Wg)


def _dispatch_body(a1_ref, a2_ref, se_ref, so_ref, be_ref):
    a1 = a1_ref[...]
    a2 = a2_ref[...]
    li = lax.broadcasted_iota(jnp.int32, (_LANES, _LANES), 0)
    lj = lax.broadcasted_iota(jnp.int32, (_LANES, _LANES), 1)
    lt_lane = (li <= lj).astype(jnp.float32)
    ri = lax.broadcasted_iota(jnp.int32, (_ROWS, _ROWS), 0)
    rj = lax.broadcasted_iota(jnp.int32, (_ROWS, _ROWS), 1)
    lt_row_strict = (rj < ri).astype(jnp.float32)

    ranks = []
    counts = []
    for e in range(_NE):
        cnt = ((a1 == e) | (a2 == e)).astype(jnp.float32)
        ic = lax.dot_general(
            cnt, lt_lane, (((1,), (0,)), ((), ())),
            precision=lax.Precision.HIGHEST,
            preferred_element_type=jnp.float32,
        )
        rs = ic[:, _LANES - 1 : _LANES]
        rp = lax.dot_general(
            lt_row_strict, rs, (((1,), (0,)), ((), ())),
            precision=lax.Precision.HIGHEST,
            preferred_element_type=jnp.float32,
        )
        rank = (ic - cnt + rp).astype(jnp.int32)
        ranks.append(rank)
        counts.append(jnp.sum(cnt).astype(jnp.int32))

    offs = []
    off = jnp.int32(0)
    ends_blk = []
    for e in range(_NE):
        offs.append(off)
        padded = ((counts[e] + (_B - 1)) // _B) * _B
        off = off + padded
        ends_blk.append(off // _B)

    se = jnp.zeros_like(a1)
    so = jnp.zeros_like(a1)
    for e in range(_NE):
        slot_e = offs[e] + ranks[e]
        se = jnp.where(a1 == e, slot_e, se)
        so = jnp.where(a2 == e, slot_e, so)
    se_ref[...] = se
    so_ref[...] = so

    bvec = lax.broadcasted_iota(jnp.int32, (1, _LANES), 1)
    be = jnp.zeros((1, _LANES), jnp.int32)
    for e in range(_NE):
        be = be + (bvec >= ends_blk[e]).astype(jnp.int32)
    be_ref[...] = jnp.minimum(be, _NE - 1)


def _dispatch(a1r, a2r):
    return pl.pallas_call(
        _dispatch_body,
        out_shape=[
            jax.ShapeDtypeStruct((_ROWS, _LANES), jnp.int32),
            jax.ShapeDtypeStruct((_ROWS, _LANES), jnp.int32),
            jax.ShapeDtypeStruct((1, _LANES), jnp.int32),
        ],
    )(a1r, a2r)


_SC_CHUNK = 32


def _make_sc_gather():
    mesh = plsc.VectorSubcoreMesh(core_axis_name="c", subcore_axis_name="s")
    info = plsc.get_sparse_core_info()
    nw = info.num_cores * info.num_subcores
    tok_per_w = _T // nw
    nck = tok_per_w // _SC_CHUNK

    @functools.partial(
        pl.kernel,
        mesh=mesh,
        out_type=[
            jax.ShapeDtypeStruct((_NS, _D), jnp.float32),
            jax.ShapeDtypeStruct((_NS, 128), jnp.float32),
        ],
        scratch_types=[
            pltpu.VMEM((2, _SC_CHUNK), jnp.int32),
            pltpu.VMEM((2, _SC_CHUNK), jnp.int32),
            pltpu.VMEM((_SC_CHUNK,), jnp.float32),
            pltpu.VMEM((_SC_CHUNK,), jnp.float32),
            pltpu.VMEM((2, _SC_CHUNK, _D), jnp.float32),
            pltpu.VMEM((2, _SC_CHUNK, 128), jnp.float32),
            pltpu.VMEM((2, _SC_CHUNK, 128), jnp.float32),
            pltpu.SemaphoreType.DMA,
            pltpu.SemaphoreType.DMA,
            pltpu.SemaphoreType.DMA,
            pltpu.SemaphoreType.DMA,
            pltpu.SemaphoreType.DMA,
            pltpu.SemaphoreType.DMA,
            pltpu.SemaphoreType.DMA,
            pltpu.SemaphoreType.DMA,
        ],
    )
    def sc_gather(x_hbm, se_hbm, so_hbm, pe_hbm, po_hbm, xs_hbm, ps_hbm,
                  idxe_v, idxo_v, pe_v, po_v, rows_v, pse_v, pso_v,
                  *sems):
        wid = lax.axis_index("s") * info.num_cores + lax.axis_index("c")
        pend = [None, None]
        for ck in range(nck):
            sl = ck % 2
            base = wid * tok_per_w + ck * _SC_CHUNK
            if pend[sl] is not None:
                for cp in pend[sl]:
                    cp.wait()
                pend[sl] = None
            pltpu.sync_copy(x_hbm.at[pl.ds(base, _SC_CHUNK)], rows_v.at[sl])
            pltpu.sync_copy(se_hbm.at[pl.ds(base, _SC_CHUNK)], idxe_v.at[sl])
            pltpu.sync_copy(so_hbm.at[pl.ds(base, _SC_CHUNK)], idxo_v.at[sl])
            pltpu.sync_copy(pe_hbm.at[pl.ds(base, _SC_CHUNK)], pe_v)
            pltpu.sync_copy(po_hbm.at[pl.ds(base, _SC_CHUNK)], po_v)
            for half in range(_SC_CHUNK // 16):
                pe_reg = pe_v[pl.ds(half * 16, 16)]
                po_reg = po_v[pl.ds(half * 16, 16)]
                for t in range(16):
                    pes = lax.squeeze(lax.slice(pe_reg, (t,), (t + 1,)), (0,))
                    pos = lax.squeeze(lax.slice(po_reg, (t,), (t + 1,)), (0,))
                    pse_v[sl, half * 16 + t, pl.ds(0, 16)] = (
                        lax.broadcast_in_dim(pes, (16,), ()))
                    pso_v[sl, half * 16 + t, pl.ds(0, 16)] = (
                        lax.broadcast_in_dim(pos, (16,), ()))
            pend[sl] = [
                pltpu.async_copy(
                    rows_v.at[sl], xs_hbm.at[idxe_v.at[sl]], sems[4 * sl]),
                pltpu.async_copy(
                    rows_v.at[sl], xs_hbm.at[idxo_v.at[sl]], sems[4 * sl + 1]),
                pltpu.async_copy(
                    pse_v.at[sl], ps_hbm.at[idxe_v.at[sl]], sems[4 * sl + 2]),
                pltpu.async_copy(
                    pso_v.at[sl], ps_hbm.at[idxo_v.at[sl]], sems[4 * sl + 3]),
            ]
        for pd in pend:
            if pd is not None:
                for cp in pd:
                    cp.wait()

    return sc_gather


def _moe_body(be_ref, xs_ref, ps_ref, w1_ref, w2_ref, w3_ref, os_ref):
    xbb = xs_ref[...].astype(jnp.bfloat16)
    h1 = lax.dot_general(
        xbb, w1_ref[0].astype(jnp.bfloat16), (((1,), (1,)), ((), ())),
        preferred_element_type=jnp.float32,
    )
    h2 = lax.dot_general(
        xbb, w2_ref[0].astype(jnp.bfloat16), (((1,), (1,)), ((), ())),
        preferred_element_type=jnp.float32,
    )
    h = (h1 * _sigmoid(h1)) * h2
    eo = lax.dot_general(
        h.astype(jnp.bfloat16), w3_ref[0].astype(jnp.bfloat16),
        (((1,), (1,)), ((), ())),
        preferred_element_type=jnp.float32,
    )
    os_ref[...] = eo * ps_ref[:, 0:1]


def _moe(be, xs, ps, w1b, w2b, w3b):
    grid_spec = pltpu.PrefetchScalarGridSpec(
        num_scalar_prefetch=1,
        grid=(_NB,),
        in_specs=[
            pl.BlockSpec((_B, _D), lambda b, be: (b, 0)),
            pl.BlockSpec((_B, 128), lambda b, be: (b, 0)),
            pl.BlockSpec((1, _H, _D), lambda b, be: (be[b], 0, 0)),
            pl.BlockSpec((1, _H, _D), lambda b, be: (be[b], 0, 0)),
            pl.BlockSpec((1, _D, _H), lambda b, be: (be[b], 0, 0)),
        ],
        out_specs=pl.BlockSpec((_B, _D), lambda b, be: (b, 0)),
    )
    return pl.pallas_call(
        _moe_body,
        grid_spec=grid_spec,
        out_shape=jax.ShapeDtypeStruct((_NS, _D), jnp.float32),
        compiler_params=pltpu.CompilerParams(
            dimension_semantics=("arbitrary",),
        ),
    )(be, xs, ps, w1b, w2b, w3b)


_CB_CHUNK = 16


def _make_sc_combine():
    mesh = plsc.VectorSubcoreMesh(core_axis_name="c", subcore_axis_name="s")
    info = plsc.get_sparse_core_info()
    nw = info.num_cores * info.num_subcores
    tok_per_w = _T // nw
    nck = tok_per_w // _CB_CHUNK

    @functools.partial(
        pl.kernel,
        mesh=mesh,
        out_type=jax.ShapeDtypeStruct((_T, _D), jnp.float32),
        scratch_types=[
            pltpu.VMEM((2, _CB_CHUNK), jnp.int32),
            pltpu.VMEM((2, _CB_CHUNK), jnp.int32),
            pltpu.VMEM((2, _CB_CHUNK, _D), jnp.float32),
            pltpu.VMEM((2, _CB_CHUNK, _D), jnp.float32),
            pltpu.SemaphoreType.DMA,
            pltpu.SemaphoreType.DMA,
            pltpu.SemaphoreType.DMA,
            pltpu.SemaphoreType.DMA,
        ],
    )
    def sc_combine(os_hbm, se_hbm, so_hbm, out_hbm,
                   idxe_v, idxo_v, re_v, ro_v, *sems):
        wid = lax.axis_index("s") * info.num_cores + lax.axis_index("c")

        def issue(ck):
            sl = ck % 2
            base = wid * tok_per_w + ck * _CB_CHUNK
            pltpu.sync_copy(se_hbm.at[pl.ds(base, _CB_CHUNK)], idxe_v.at[sl])
            pltpu.sync_copy(so_hbm.at[pl.ds(base, _CB_CHUNK)], idxo_v.at[sl])
            return [
                pltpu.async_copy(
                    os_hbm.at[idxe_v.at[sl]], re_v.at[sl], sems[2 * sl]),
                pltpu.async_copy(
                    os_hbm.at[idxo_v.at[sl]], ro_v.at[sl], sems[2 * sl + 1]),
            ]

        pend = issue(0)
        for ck in range(nck):
            sl = ck % 2
            for cp in pend:
                cp.wait()
            if ck + 1 < nck:
                pend = issue(ck + 1)

            def abody(i, _, sl=sl):
                t = i // (_D // 16)
                dd = i % (_D // 16)
                ds16 = pl.ds(dd * 16, 16)
                re_v[sl, t, ds16] += ro_v[sl, t, ds16]
                return 0

            lax.fori_loop(0, _CB_CHUNK * (_D // 16), abody, 0, unroll=16)
            base = wid * tok_per_w + ck * _CB_CHUNK
            pltpu.sync_copy(re_v.at[sl], out_hbm.at[pl.ds(base, _CB_CHUNK)])

    return sc_combine


def kernel(x, Wg, W1, W2, W3):
    b, s, d = x.shape
    xf = x.reshape(b * s, d)

    a1, a2, p1, p2 = _router(xf, Wg)
    a1r = a1.reshape(_ROWS, _LANES)
    a2r = a2.reshape(_ROWS, _LANES)
    se, so, be = _dispatch(a1r, a2r)
    se = se.reshape(_T)
    so = so.reshape(_T)
    be = be.reshape(_LANES)[:_NB]

    xs, ps = _make_sc_gather()(xf, se, so, p1.reshape(_T), p2.reshape(_T))
    os = _moe(be, xs, ps, W1, W2, W3)
    out = _make_sc_combine()(os, se, so)
    return out.reshape(b, s, d)
